# Initial kernel scaffold; baseline (speedup 1.0000x reference)
import jax
import jax.numpy as jnp
from jax.experimental import pallas as pl


def kernel(x, edge_index, W1, b1, W2, b2):
    raise NotImplementedError("write your pallas kernel here")


# speedup vs baseline: 22.8436x; 22.8436x over previous
import functools

import jax
import jax.numpy as jnp
from jax import lax
from jax.experimental import pallas as pl
from jax.experimental.pallas import tpu as pltpu
from jax.experimental.pallas import tpu_sc as plsc

N_NODES = 10000
N_PAD = 10240
N_EDGES = 320000
IN_CH = 128
HID = 256
OUT_CH = 128

NC = 2
NS = 16
NW = NC * NS
EW = N_EDGES // NW
CHUNK = 125
NCHUNK = EW // CHUNK

ROWS_PER_TILE = N_PAD // NS

_MESH = plsc.VectorSubcoreMesh(core_axis_name="c", subcore_axis_name="s")


@functools.partial(
    pl.kernel,
    out_type=jax.ShapeDtypeStruct((NC, N_PAD), jnp.float32),
    mesh=_MESH,
    scratch_types=[
        pltpu.VMEM((NCHUNK, CHUNK), jnp.int32),
        pltpu.VMEM((CHUNK,), jnp.float32),
        pltpu.VMEM_SHARED((N_PAD,), jnp.float32),
        pltpu.SemaphoreType.DMA,
    ],
)
def _sc_degree(dst_hbm, ones_hbm, zeros1_hbm, degp_hbm, dst_v, ones_v,
               deg_sh, sem):
    c = lax.axis_index("c")
    s = lax.axis_index("s")
    w = c * NS + s
    r0 = s * ROWS_PER_TILE
    pltpu.sync_copy(zeros1_hbm.at[pl.ds(r0, ROWS_PER_TILE)],
                    deg_sh.at[pl.ds(r0, ROWS_PER_TILE)])
    pltpu.sync_copy(ones_hbm, ones_v)
    pltpu.sync_copy(dst_hbm.at[w], dst_v)
    plsc.subcore_barrier()

    def body(i, carry):
        pltpu.sync_copy(ones_v, deg_sh.at[dst_v.at[i]], add=True)
        return carry

    lax.fori_loop(0, NCHUNK, body, 0)
    plsc.subcore_barrier()
    pltpu.sync_copy(deg_sh.at[pl.ds(r0, ROWS_PER_TILE)],
                    degp_hbm.at[c, pl.ds(r0, ROWS_PER_TILE)])


@functools.partial(
    pl.kernel,
    out_type=jax.ShapeDtypeStruct((NC, N_PAD, IN_CH), jnp.float32),
    mesh=_MESH,
    scratch_types=[
        pltpu.VMEM((NCHUNK, CHUNK), jnp.int32),
        pltpu.VMEM((NCHUNK, CHUNK), jnp.int32),
        pltpu.VMEM((CHUNK, IN_CH), jnp.float32),
        pltpu.VMEM_SHARED((N_PAD, IN_CH), jnp.float32),
        pltpu.SemaphoreType.DMA,
    ],
)
def _sc_aggregate(y_hbm, src_hbm, dst_hbm, zeros_hbm, zp_hbm, src_v, dst_v,
                  rows_v, z_sh, sem):
    c = lax.axis_index("c")
    s = lax.axis_index("s")
    w = c * NS + s
    r0 = s * ROWS_PER_TILE
    pltpu.sync_copy(zeros_hbm.at[pl.ds(r0, ROWS_PER_TILE)],
                    z_sh.at[pl.ds(r0, ROWS_PER_TILE)])
    pltpu.sync_copy(src_hbm.at[w], src_v)
    pltpu.sync_copy(dst_hbm.at[w], dst_v)
    plsc.subcore_barrier()

    def body(i, carry):
        pltpu.async_copy(y_hbm.at[src_v.at[i]], rows_v, sem).wait()
        pltpu.sync_copy(rows_v, z_sh.at[dst_v.at[i]], add=True)
        return carry

    lax.fori_loop(0, NCHUNK, body, 0)
    plsc.subcore_barrier()
    pltpu.sync_copy(z_sh.at[pl.ds(r0, ROWS_PER_TILE)],
                    zp_hbm.at[c, pl.ds(r0, ROWS_PER_TILE)])


_BLK = 1024
_GRID = N_PAD // _BLK


def _t0_body(dp0_ref, dp1_ref, x_ref, d_ref, y_ref):
    deg = dp0_ref[...] + dp1_ref[...] + 1.0
    d = lax.rsqrt(deg)
    d_ref[...] = d
    y_ref[...] = x_ref[...] * d


def _tc_scale(degp, x_pad):
    dp0 = degp[0].reshape(N_PAD, 1)
    dp1 = degp[1].reshape(N_PAD, 1)
    row = lambda i: (i, 0)
    return pl.pallas_call(
        _t0_body,
        grid=(_GRID,),
        in_specs=[
            pl.BlockSpec((_BLK, 1), row),
            pl.BlockSpec((_BLK, 1), row),
            pl.BlockSpec((_BLK, IN_CH), row),
        ],
        out_specs=[
            pl.BlockSpec((_BLK, 1), row),
            pl.BlockSpec((_BLK, IN_CH), row),
        ],
        out_shape=[
            jax.ShapeDtypeStruct((N_PAD, 1), jnp.float32),
            jax.ShapeDtypeStruct((N_PAD, IN_CH), jnp.float32),
        ],
    )(dp0, dp1, x_pad)


def _t1_body(za_ref, zb_ref, y1_ref, d_ref, w1_ref, b1_ref, w2_ref, y2_ref):
    d = d_ref[...]
    m = (za_ref[...] + zb_ref[...] + y1_ref[...]) * d
    h = jnp.dot(m, w1_ref[...], preferred_element_type=jnp.float32,
                precision=lax.Precision.HIGHEST)
    h = jnp.maximum(h + b1_ref[...], 0.0)
    g = jnp.dot(h, w2_ref[...], preferred_element_type=jnp.float32,
                precision=lax.Precision.HIGHEST)
    y2_ref[...] = g * d


def _tc_mid(zp, y1, d, W1, b1, W2):
    row = lambda i: (i, 0)
    full = lambda i: (0, 0)
    return pl.pallas_call(
        _t1_body,
        grid=(_GRID,),
        in_specs=[
            pl.BlockSpec((_BLK, IN_CH), row),
            pl.BlockSpec((_BLK, IN_CH), row),
            pl.BlockSpec((_BLK, IN_CH), row),
            pl.BlockSpec((_BLK, 1), row),
            pl.BlockSpec((IN_CH, HID), full),
            pl.BlockSpec((1, HID), full),
            pl.BlockSpec((HID, OUT_CH), full),
        ],
        out_specs=pl.BlockSpec((_BLK, OUT_CH), row),
        out_shape=jax.ShapeDtypeStruct((N_PAD, OUT_CH), jnp.float32),
    )(zp[0], zp[1], y1, d, W1, b1.reshape(1, HID), W2)


def _t2_body(za_ref, zb_ref, y2_ref, d_ref, b2_ref, out_ref):
    u = (za_ref[...] + zb_ref[...] + y2_ref[...]) * d_ref[...]
    out_ref[...] = u + b2_ref[...]


def _tc_final(zp, y2, d, b2):
    row = lambda i: (i, 0)
    full = lambda i: (0, 0)
    return pl.pallas_call(
        _t2_body,
        grid=(_GRID,),
        in_specs=[
            pl.BlockSpec((_BLK, OUT_CH), row),
            pl.BlockSpec((_BLK, OUT_CH), row),
            pl.BlockSpec((_BLK, OUT_CH), row),
            pl.BlockSpec((_BLK, 1), row),
            pl.BlockSpec((1, OUT_CH), full),
        ],
        out_specs=pl.BlockSpec((_BLK, OUT_CH), row),
        out_shape=jax.ShapeDtypeStruct((N_PAD, OUT_CH), jnp.float32),
    )(zp[0], zp[1], y2, d, b2.reshape(1, OUT_CH))


@jax.jit
def kernel(x, edge_index, W1, b1, W2, b2):
    src3 = edge_index[0].astype(jnp.int32).reshape(NW, NCHUNK, CHUNK)
    dst3 = edge_index[1].astype(jnp.int32).reshape(NW, NCHUNK, CHUNK)
    x_pad = jnp.pad(x, ((0, N_PAD - N_NODES), (0, 0)))
    ones = jnp.ones((CHUNK,), jnp.float32)
    zeros1 = jnp.zeros((N_PAD,), jnp.float32)
    zeros128 = jnp.zeros((N_PAD, IN_CH), jnp.float32)

    degp = _sc_degree(dst3, ones, zeros1)
    d, y1 = _tc_scale(degp, x_pad)
    z1p = _sc_aggregate(y1, src3, dst3, zeros128)
    y2 = _tc_mid(z1p, y1, d, W1, b1, W2)
    z2p = _sc_aggregate(y2, src3, dst3, zeros128)
    out = _tc_final(z2p, y2, d, b2)
    return out[:N_NODES]


# speedup vs baseline: 28.1954x; 1.2343x over previous
import functools

import jax
import jax.numpy as jnp
from jax import lax
from jax.experimental import pallas as pl
from jax.experimental.pallas import tpu as pltpu
from jax.experimental.pallas import tpu_sc as plsc

N_NODES = 10000
N_PAD = 10240
N_EDGES = 320000
IN_CH = 128
HID = 256
OUT_CH = 128

NC = 2
NS = 16
NW = NC * NS
EW = N_EDGES // NW
CHUNK = 125
NCHUNK = EW // CHUNK

ROWS_PER_TILE = N_PAD // NS

_MESH = plsc.VectorSubcoreMesh(core_axis_name="c", subcore_axis_name="s")


@functools.partial(
    pl.kernel,
    out_type=jax.ShapeDtypeStruct((NC, N_PAD), jnp.float32),
    mesh=_MESH,
    scratch_types=[
        pltpu.VMEM((NCHUNK, CHUNK), jnp.int32),
        pltpu.VMEM((CHUNK,), jnp.float32),
        pltpu.VMEM_SHARED((N_PAD,), jnp.float32),
        pltpu.SemaphoreType.DMA,
    ],
)
def _sc_degree(dst_hbm, ones_hbm, zeros1_hbm, degp_hbm, dst_v, ones_v,
               deg_sh, sem):
    c = lax.axis_index("c")
    s = lax.axis_index("s")
    w = c * NS + s
    r0 = s * ROWS_PER_TILE
    pltpu.sync_copy(zeros1_hbm.at[pl.ds(r0, ROWS_PER_TILE)],
                    deg_sh.at[pl.ds(r0, ROWS_PER_TILE)])
    pltpu.sync_copy(ones_hbm, ones_v)
    pltpu.sync_copy(dst_hbm.at[w], dst_v)
    plsc.subcore_barrier()

    def body(i, carry):
        pltpu.sync_copy(ones_v, deg_sh.at[dst_v.at[i]], add=True)
        return carry

    lax.fori_loop(0, NCHUNK, body, 0)
    plsc.subcore_barrier()
    pltpu.sync_copy(deg_sh.at[pl.ds(r0, ROWS_PER_TILE)],
                    degp_hbm.at[c, pl.ds(r0, ROWS_PER_TILE)])


@functools.partial(
    pl.kernel,
    out_type=jax.ShapeDtypeStruct((NC, N_PAD, IN_CH), jnp.float32),
    mesh=_MESH,
    scratch_types=[
        pltpu.VMEM((2, CHUNK), jnp.int32),
        pltpu.VMEM((2, CHUNK), jnp.int32),
        pltpu.VMEM((CHUNK, IN_CH), jnp.float32),
        pltpu.VMEM((CHUNK, IN_CH), jnp.float32),
        pltpu.VMEM_SHARED((N_PAD, IN_CH), jnp.float32),
        pltpu.SemaphoreType.DMA,
        pltpu.SemaphoreType.DMA,
        pltpu.SemaphoreType.DMA,
        pltpu.SemaphoreType.DMA,
    ],
)
def _sc_aggregate(y_hbm, idx_hbm, zeros_hbm, zp_hbm, idx0_v, idx1_v,
                  rows0_v, rows1_v, z_sh, gsem0, gsem1, isem0, isem1):
    c = lax.axis_index("c")
    s = lax.axis_index("s")
    w = c * NS + s
    r0 = s * ROWS_PER_TILE
    pltpu.sync_copy(zeros_hbm.at[pl.ds(r0, ROWS_PER_TILE)],
                    z_sh.at[pl.ds(r0, ROWS_PER_TILE)])
    idx = (idx0_v, idx1_v)
    rows = (rows0_v, rows1_v)
    gsems = (gsem0, gsem1)
    isems = (isem0, isem1)
    pltpu.sync_copy(idx_hbm.at[w, 0], idx[0])
    pltpu.async_copy(idx_hbm.at[w, 1], idx[1], isems[1])
    plsc.subcore_barrier()
    pltpu.async_copy(y_hbm.at[idx[0].at[0]], rows[0], gsems[0])

    @pl.loop(0, NCHUNK, step=2)
    def _loop(i):
        for b in range(2):
            j = i + b
            nb = 1 - b
            pltpu.make_async_copy(y_hbm.at[idx[b].at[0]], rows[b],
                                  gsems[b]).wait()

            @pl.when(j + 1 < NCHUNK)
            def _():
                pltpu.make_async_copy(idx_hbm.at[w, j + 1], idx[nb],
                                      isems[nb]).wait()
                pltpu.async_copy(y_hbm.at[idx[nb].at[0]], rows[nb], gsems[nb])

            pltpu.sync_copy(rows[b], z_sh.at[idx[b].at[1]], add=True)

            @pl.when(j + 2 < NCHUNK)
            def _():
                pltpu.async_copy(idx_hbm.at[w, j + 2], idx[b], isems[b])

    plsc.subcore_barrier()
    pltpu.sync_copy(z_sh.at[pl.ds(r0, ROWS_PER_TILE)],
                    zp_hbm.at[c, pl.ds(r0, ROWS_PER_TILE)])


_BLK = 1024
_GRID = N_PAD // _BLK


def _t0_body(dp0_ref, dp1_ref, x_ref, d_ref, y_ref):
    deg = dp0_ref[...] + dp1_ref[...] + 1.0
    d = lax.rsqrt(deg)
    d_ref[...] = d
    y_ref[...] = x_ref[...] * d


def _tc_scale(degp, x_pad):
    dp0 = degp[0].reshape(N_PAD, 1)
    dp1 = degp[1].reshape(N_PAD, 1)
    row = lambda i: (i, 0)
    return pl.pallas_call(
        _t0_body,
        grid=(_GRID,),
        in_specs=[
            pl.BlockSpec((_BLK, 1), row),
            pl.BlockSpec((_BLK, 1), row),
            pl.BlockSpec((_BLK, IN_CH), row),
        ],
        out_specs=[
            pl.BlockSpec((_BLK, 1), row),
            pl.BlockSpec((_BLK, IN_CH), row),
        ],
        out_shape=[
            jax.ShapeDtypeStruct((N_PAD, 1), jnp.float32),
            jax.ShapeDtypeStruct((N_PAD, IN_CH), jnp.float32),
        ],
    )(dp0, dp1, x_pad)


def _t1_body(za_ref, zb_ref, y1_ref, d_ref, w1_ref, b1_ref, w2_ref, y2_ref):
    d = d_ref[...]
    m = (za_ref[...] + zb_ref[...] + y1_ref[...]) * d
    h = jnp.dot(m, w1_ref[...], preferred_element_type=jnp.float32,
                precision=lax.Precision.HIGHEST)
    h = jnp.maximum(h + b1_ref[...], 0.0)
    g = jnp.dot(h, w2_ref[...], preferred_element_type=jnp.float32,
                precision=lax.Precision.HIGHEST)
    y2_ref[...] = g * d


def _tc_mid(zp, y1, d, W1, b1, W2):
    row = lambda i: (i, 0)
    full = lambda i: (0, 0)
    return pl.pallas_call(
        _t1_body,
        grid=(_GRID,),
        in_specs=[
            pl.BlockSpec((_BLK, IN_CH), row),
            pl.BlockSpec((_BLK, IN_CH), row),
            pl.BlockSpec((_BLK, IN_CH), row),
            pl.BlockSpec((_BLK, 1), row),
            pl.BlockSpec((IN_CH, HID), full),
            pl.BlockSpec((1, HID), full),
            pl.BlockSpec((HID, OUT_CH), full),
        ],
        out_specs=pl.BlockSpec((_BLK, OUT_CH), row),
        out_shape=jax.ShapeDtypeStruct((N_PAD, OUT_CH), jnp.float32),
    )(zp[0], zp[1], y1, d, W1, b1.reshape(1, HID), W2)


def _t2_body(za_ref, zb_ref, y2_ref, d_ref, b2_ref, out_ref):
    u = (za_ref[...] + zb_ref[...] + y2_ref[...]) * d_ref[...]
    out_ref[...] = u + b2_ref[...]


def _tc_final(zp, y2, d, b2):
    row = lambda i: (i, 0)
    full = lambda i: (0, 0)
    return pl.pallas_call(
        _t2_body,
        grid=(_GRID,),
        in_specs=[
            pl.BlockSpec((_BLK, OUT_CH), row),
            pl.BlockSpec((_BLK, OUT_CH), row),
            pl.BlockSpec((_BLK, OUT_CH), row),
            pl.BlockSpec((_BLK, 1), row),
            pl.BlockSpec((1, OUT_CH), full),
        ],
        out_specs=pl.BlockSpec((_BLK, OUT_CH), row),
        out_shape=jax.ShapeDtypeStruct((N_PAD, OUT_CH), jnp.float32),
    )(zp[0], zp[1], y2, d, b2.reshape(1, OUT_CH))


@jax.jit
def kernel(x, edge_index, W1, b1, W2, b2):
    ei = edge_index.astype(jnp.int32)
    dst3 = ei[1].reshape(NW, NCHUNK, CHUNK)
    packed = ei.reshape(2, NW, NCHUNK, CHUNK).transpose(1, 2, 0, 3)
    x_pad = jnp.pad(x, ((0, N_PAD - N_NODES), (0, 0)))
    ones = jnp.ones((CHUNK,), jnp.float32)
    zeros1 = jnp.zeros((N_PAD,), jnp.float32)
    zeros128 = jnp.zeros((N_PAD, IN_CH), jnp.float32)

    degp = _sc_degree(dst3, ones, zeros1)
    d, y1 = _tc_scale(degp, x_pad)
    z1p = _sc_aggregate(y1, packed, zeros128)
    y2 = _tc_mid(z1p, y1, d, W1, b1, W2)
    z2p = _sc_aggregate(y2, packed, zeros128)
    out = _tc_final(z2p, y2, d, b2)
    return out[:N_NODES]
